# Initial kernel scaffold; baseline (speedup 1.0000x reference)
import jax
import jax.numpy as jnp
from jax.experimental import pallas as pl


def kernel(utterances, w0x, whx, wh, b, wlin, blin):
    raise NotImplementedError("write your pallas kernel here")


# speedup vs baseline: 2.0176x; 2.0176x over previous
from functools import partial

import jax
import jax.numpy as jnp
from jax.experimental import pallas as pl
from jax.experimental.pallas import tpu as pltpu


def _wavefront_kernel(x_ref, w0x_ref, wh0_ref, wx1_ref, wh1_ref, wx2_ref,
                      wh2_ref, b_ref, wlin_ref, blin_ref, out_ref,
                      h0, c0, h1, c1, h2, c2, *, hidden, total_frames):
    H, T = hidden, total_frames
    f32 = jnp.float32

    b0 = b_ref[0:1, :]
    b1 = b_ref[1:2, :]
    b2 = b_ref[2:3, :]

    def cell(pre, c):
        sig = jax.nn.sigmoid(pre[:, :3 * H])
        g_g = jnp.tanh(pre[:, 3 * H:])
        i_g = sig[:, 0 * H:1 * H]
        f_g = sig[:, 1 * H:2 * H]
        o_g = sig[:, 2 * H:3 * H]
        c_new = f_g * c + i_g * g_g
        h_new = (o_g * jnp.tanh(c_new)).astype(jnp.bfloat16)
        return h_new, c_new

    def layer_step(inp, wx, wh_r, bias, h_r, c_r):
        pre = (jnp.dot(inp, wx, preferred_element_type=f32)
               + jnp.dot(h_r[...], wh_r[...], preferred_element_type=f32)
               + bias)
        h_new, c_new = cell(pre, c_r[...])
        h_r[...] = h_new
        c_r[...] = c_new

    for r in (h0, h1, h2, c0, c1, c2):
        r[...] = jnp.zeros_like(r)

    layer_step(x_ref[0], w0x_ref[...], wh0_ref, b0, h0, c0)
    h0_prev = h0[...]
    layer_step(x_ref[1], w0x_ref[...], wh0_ref, b0, h0, c0)
    layer_step(h0_prev, wx1_ref[...], wh1_ref, b1, h1, c1)

    def body(s, carry):
        h0_prev = h0[...]
        h1_prev = h1[...]
        layer_step(x_ref[s], w0x_ref[...], wh0_ref, b0, h0, c0)
        layer_step(h0_prev, wx1_ref[...], wh1_ref, b1, h1, c1)
        layer_step(h1_prev, wx2_ref[...], wh2_ref, b2, h2, c2)
        return carry

    jax.lax.fori_loop(2, T, body, 0, unroll=4)

    h0_prev = h0[...]
    h1_prev = h1[...]
    layer_step(h0_prev, wx1_ref[...], wh1_ref, b1, h1, c1)
    layer_step(h1_prev, wx2_ref[...], wh2_ref, b2, h2, c2)
    h1_prev = h1[...]
    layer_step(h1_prev, wx2_ref[...], wh2_ref, b2, h2, c2)

    y = jnp.dot(h2[...], wlin_ref[...], preferred_element_type=f32)
    y = jnp.maximum(y + blin_ref[...], 0.0)
    ssq = jnp.sum(y * y, axis=1, keepdims=True)
    out_ref[...] = y * jax.lax.rsqrt(jnp.maximum(ssq, 1e-12))


def kernel(utterances, w0x, whx, wh, b, wlin, blin):
    B, T, D_in = utterances.shape
    H = wh.shape[1]
    E = wlin.shape[1]
    L = b.shape[0]
    assert L == 3, "wavefront kernel is specialized to 3 LSTM layers"

    Dp = ((D_in + 127) // 128) * 128
    nb = 2
    Bb = -(-B // (8 * nb)) * 8
    B_pad = nb * Bb

    x = jnp.transpose(utterances.astype(jnp.bfloat16), (1, 0, 2))
    x = jnp.pad(x, ((0, 0), (0, B_pad - B), (0, Dp - D_in)))
    x = x.reshape(T, nb, Bb, Dp).transpose(1, 0, 2, 3)

    w0xp = jnp.pad(w0x, ((0, Dp - D_in), (0, 0)))

    kernel_fn = partial(_wavefront_kernel, hidden=H, total_frames=T)

    full = lambda bi: (0, 0)
    out = pl.pallas_call(
        kernel_fn,
        out_shape=jax.ShapeDtypeStruct((B_pad, E), jnp.float32),
        grid=(nb,),
        in_specs=[
            pl.BlockSpec((None, T, Bb, Dp), lambda bi: (bi, 0, 0, 0)),
            pl.BlockSpec((Dp, 4 * H), full),
            pl.BlockSpec((None, H, 4 * H), lambda bi: (0, 0, 0)),
            pl.BlockSpec((None, H, 4 * H), lambda bi: (0, 0, 0)),
            pl.BlockSpec((None, H, 4 * H), lambda bi: (1, 0, 0)),
            pl.BlockSpec((None, H, 4 * H), lambda bi: (1, 0, 0)),
            pl.BlockSpec((None, H, 4 * H), lambda bi: (2, 0, 0)),
            pl.BlockSpec((L, 4 * H), full),
            pl.BlockSpec((H, E), full),
            pl.BlockSpec((1, E), full),
        ],
        out_specs=pl.BlockSpec((Bb, E), lambda bi: (bi, 0)),
        scratch_shapes=[
            pltpu.VMEM((Bb, H), jnp.bfloat16),
            pltpu.VMEM((Bb, H), jnp.float32),
            pltpu.VMEM((Bb, H), jnp.bfloat16),
            pltpu.VMEM((Bb, H), jnp.float32),
            pltpu.VMEM((Bb, H), jnp.bfloat16),
            pltpu.VMEM((Bb, H), jnp.float32),
        ],
        compiler_params=pltpu.CompilerParams(
            dimension_semantics=("parallel",)),
    )(x, w0xp, wh, whx, wh, whx, wh, b, wlin, blin)
    return out[:B]
